# baseline (speedup 1.0000x reference)
import functools

import jax
import jax.numpy as jnp
from jax import lax
from jax.experimental import pallas as pl
from jax.experimental.pallas import tpu as pltpu
from jax.experimental.pallas import tpu_sc as plsc

NT = 32768
NE = 64
NC = 2
NS = 16
NW = NC * NS
TPW = NT // NW
CHUNK = 256
NCHUNKS = TPW // CHUNK
L = 16
NV = NE // L
WPT = NE * L // NS

_mesh = plsc.VectorSubcoreMesh(core_axis_name="c", subcore_axis_name="s")


@functools.partial(
    pl.kernel,
    out_type=[
        jax.ShapeDtypeStruct((NT,), jnp.int32),
        jax.ShapeDtypeStruct((NT,), jnp.float32),
        jax.ShapeDtypeStruct((NC * NE,), jnp.int32),
        jax.ShapeDtypeStruct((NC * NE * L,), jnp.float32),
    ],
    mesh=_mesh,
    compiler_params=pltpu.CompilerParams(needs_layout_passes=False),
    scratch_types=[
        pltpu.VMEM((CHUNK * NE,), jnp.float32),
        pltpu.VMEM((CHUNK * NE,), jnp.float32),
        pltpu.VMEM((CHUNK * NE,), jnp.float32),
        pltpu.VMEM((NE * L,), jnp.float32),
        pltpu.VMEM((TPW,), jnp.int32),
        pltpu.VMEM((TPW,), jnp.float32),
        pltpu.VMEM((NE,), jnp.int32),
        pltpu.VMEM((NS * WPT,), jnp.float32),
        pltpu.VMEM((NS * NE,), jnp.int32),
        pltpu.VMEM((NE,), jnp.float32),
        pltpu.VMEM_SHARED((NS * NE * L,), jnp.float32),
        pltpu.VMEM_SHARED((NS * NE,), jnp.int32),
        pltpu.SemaphoreType.DMA,
        pltpu.SemaphoreType.DMA,
    ],
)
def _gate_kernel(x_hbm, sample_hbm, mult_hbm, cnt_hbm, psum_hbm,
                 buf0, buf1, tbuf, psum_t, sample_buf, mult_buf, cnt_buf,
                 agg_ps, agg_ct, stage, sh_ps, sh_ct, sem0, sem1):
    cid = lax.axis_index("c")
    sid = lax.axis_index("s")
    wid = cid * NS + sid
    tok0 = wid * TPW

    idx0 = lax.iota(jnp.int32, L)
    ones_i = jnp.ones((L,), jnp.int32)
    z16f = jnp.zeros((L,), jnp.float32)
    z16i = jnp.zeros((L,), jnp.int32)
    bufs = (buf0, buf1)
    sems = (sem0, sem1)

    cps = [None] * NCHUNKS
    cps[0] = pltpu.async_copy(
        x_hbm.at[pl.ds(tok0 * NE, CHUNK * NE)], bufs[0], sems[0])

    for e in range(NE):
        psum_t[pl.ds(e * L, L)] = z16f
    for j in range(NV):
        cnt_buf[pl.ds(j * L, L)] = z16i

    for k in range(NCHUNKS):
        b = k % 2
        if k + 1 < NCHUNKS:
            cps[k + 1] = pltpu.async_copy(
                x_hbm.at[pl.ds((tok0 + (k + 1) * CHUNK) * NE, CHUNK * NE)],
                bufs[1 - b], sems[1 - b])
        cps[k].wait()
        buf = bufs[b]

        def gbody(g, k=k, buf=buf):
            gidx = g * (L * NE) + idx0 * NE
            tb = g * (L * NE)
            ms, ams = [], []
            for cix in range(NV):
                e0 = cix * (NE // NV)
                m = plsc.load_gather(buf, [gidx + e0])
                tbuf[pl.ds(tb + e0 * L, L)] = m
                am = jnp.full((L,), e0, jnp.int32)
                for e in range(e0 + 1, e0 + NE // NV):
                    v = plsc.load_gather(buf, [gidx + e])
                    tbuf[pl.ds(tb + e * L, L)] = v
                    am = jnp.where(v > m, jnp.int32(e), am)
                    m = jnp.maximum(m, v)
                ms.append(m)
                ams.append(am)
            mm, amax = ms[0], ams[0]
            for cix in range(1, NV):
                amax = jnp.where(ms[cix] > mm, ams[cix], amax)
                mm = jnp.maximum(mm, ms[cix])
            ss = [z16f] * NV
            for cix in range(NV):
                e0 = cix * (NE // NV)
                for e in range(e0, e0 + NE // NV):
                    ex = jnp.exp(tbuf[pl.ds(tb + e * L, L)] - mm)
                    tbuf[pl.ds(tb + e * L, L)] = ex
                    ss[cix] = ss[cix] + ex
            r = 1.0 / ((ss[0] + ss[1]) + (ss[2] + ss[3]))
            off = k * CHUNK + g * L
            sample_buf[pl.ds(off, L)] = amax
            mult_buf[pl.ds(off, L)] = r
            plsc.addupdate_scatter(cnt_buf, [amax], ones_i)
            for e in range(NE):
                p = tbuf[pl.ds(tb + e * L, L)] * r
                plsc.addupdate(psum_t.at[pl.ds(e * L, L)], p)

        plsc.parallel_loop(0, CHUNK // L, 1, unroll=2)(gbody)

    pltpu.sync_copy(sample_buf, sample_hbm.at[pl.ds(tok0, TPW)])
    pltpu.sync_copy(mult_buf, mult_hbm.at[pl.ds(tok0, TPW)])

    pltpu.sync_copy(psum_t, sh_ps.at[pl.ds(sid * NE * L, NE * L)])
    pltpu.sync_copy(cnt_buf, sh_ct.at[pl.ds(sid * NE, NE)])
    plsc.subcore_barrier()

    for rr in range(NS):
        pltpu.sync_copy(sh_ps.at[pl.ds(rr * NE * L + sid * WPT, WPT)],
                        agg_ps.at[pl.ds(rr * WPT, WPT)])
    accp = [z16f for _ in range(WPT // L)]
    for rr in range(NS):
        for j in range(WPT // L):
            accp[j] = accp[j] + agg_ps[pl.ds(rr * WPT + j * L, L)]
    for j in range(WPT // L):
        stage[pl.ds(j * L, L)] = accp[j]
    pltpu.sync_copy(stage.at[pl.ds(0, WPT)],
                    psum_hbm.at[pl.ds(cid * NE * L + sid * WPT, WPT)])

    @pl.when(sid == 0)
    def _():
        pltpu.sync_copy(sh_ct, agg_ct)
        accc = [z16i for _ in range(NV)]
        for rr in range(NS):
            for j in range(NV):
                accc[j] = accc[j] + agg_ct[pl.ds(rr * NE + j * L, L)]
        for j in range(NV):
            cnt_buf[pl.ds(j * L, L)] = accc[j]
        pltpu.sync_copy(cnt_buf, cnt_hbm.at[pl.ds(cid * NE, NE)])


def _loss_body(cnt_ref, ps_ref, out_ref):
    cntf = cnt_ref[...].astype(jnp.float32)
    ps = ps_ref[...]
    f2 = jnp.sum(cntf, axis=0, keepdims=True) * (1.0 / NT)
    pm2 = jnp.sum(ps, axis=0, keepdims=True) * (1.0 / NT)
    out_ref[...] = jnp.float32(NE) * jnp.sum(pm2 * f2, axis=1, keepdims=True)


def kernel(logits):
    x = logits.reshape(-1)
    sample, mult, cnt, psflat = _gate_kernel(x)
    ps = psflat.reshape(NC, NE, L).transpose(0, 2, 1).reshape(NC * L, NE)
    loss = pl.pallas_call(
        _loss_body,
        out_shape=jax.ShapeDtypeStruct((1, 1), jnp.float32),
    )(cnt.reshape(NC, NE), ps)
    return sample, mult.reshape(NT, 1), loss.reshape(())


# speedup vs baseline: 1.3066x; 1.3066x over previous
import functools

import jax
import jax.numpy as jnp
from jax import lax
from jax.experimental import pallas as pl
from jax.experimental.pallas import tpu as pltpu
from jax.experimental.pallas import tpu_sc as plsc

NT = 32768
NE = 64
NC = 2
NS = 16
NW = NC * NS
TPW = NT // NW
CHUNK = 64
NPAIRS = TPW // (2 * CHUNK)
L = 16
NV = NE // L
CW = CHUNK * NE

_mesh = plsc.VectorSubcoreMesh(core_axis_name="c", subcore_axis_name="s")


@functools.partial(
    pl.kernel,
    out_type=[
        jax.ShapeDtypeStruct((NT,), jnp.int32),
        jax.ShapeDtypeStruct((NT,), jnp.float32),
        jax.ShapeDtypeStruct((NC * NE,), jnp.int32),
        jax.ShapeDtypeStruct((NC * NE,), jnp.float32),
    ],
    mesh=_mesh,
    compiler_params=pltpu.CompilerParams(needs_layout_passes=False),
    scratch_types=[
        pltpu.VMEM((CW,), jnp.float32),
        pltpu.VMEM((CW,), jnp.float32),
        pltpu.VMEM((TPW,), jnp.int32),
        pltpu.VMEM((TPW,), jnp.float32),
        pltpu.VMEM((NE,), jnp.int32),
        pltpu.VMEM((NE,), jnp.float32),
        pltpu.VMEM((NS * NE,), jnp.float32),
        pltpu.VMEM((NS * NE,), jnp.int32),
        pltpu.VMEM_SHARED((NS * NE,), jnp.float32),
        pltpu.VMEM_SHARED((NS * NE,), jnp.int32),
        pltpu.SemaphoreType.DMA,
        pltpu.SemaphoreType.DMA,
    ],
)
def _gate_kernel(x_hbm, sample_hbm, mult_hbm, cnt_hbm, psum_hbm,
                 buf0, buf1, sample_buf, mult_buf, cnt_buf, psum_buf,
                 agg_ps, agg_ct, sh_ps, sh_ct, sem0, sem1):
    cid = lax.axis_index("c")
    sid = lax.axis_index("s")
    wid = cid * NS + sid
    tok0 = wid * TPW
    word0 = tok0 * NE
    last_off = word0 + TPW * NE - CW

    idx0 = lax.iota(jnp.int32, L)
    ones_i = jnp.ones((L,), jnp.int32)
    z16f = jnp.zeros((L,), jnp.float32)
    z16i = jnp.zeros((L,), jnp.int32)
    idxc = [idx0 + j * L for j in range(NV)]
    lmask = [idx0 == t for t in range(L)]
    big_i = jnp.full((L,), NE, jnp.int32)

    def process(buf, out_off, ps):
        ps = list(ps)
        for blk in range(CHUNK // L):
            svec = z16i
            mvec = z16f
            for tt in range(L):
                t = blk * L + tt
                l = [buf[pl.ds(t * NE + j * L, L)] for j in range(NV)]
                m = jnp.max(jnp.maximum(jnp.maximum(l[0], l[1]),
                                        jnp.maximum(l[2], l[3])))
                ex = [jnp.exp(l[j] - m) for j in range(NV)]
                s = jnp.sum((ex[0] + ex[1]) + (ex[2] + ex[3]))
                r = 1.0 / jnp.broadcast_to(s, (L,))
                c = [jnp.where(l[j] == m, idxc[j], big_i) for j in range(NV)]
                samp = jnp.min(jnp.minimum(jnp.minimum(c[0], c[1]),
                                           jnp.minimum(c[2], c[3])))
                for j in range(NV):
                    ps[j] = ps[j] + ex[j] * r
                svec = jnp.where(lmask[tt], samp, svec)
                mvec = jnp.where(lmask[tt], r, mvec)
            sample_buf[pl.ds(out_off + blk * L, L)] = svec
            mult_buf[pl.ds(out_off + blk * L, L)] = mvec
            plsc.addupdate_scatter(cnt_buf, [svec], ones_i)
        return ps

    for j in range(NV):
        cnt_buf[pl.ds(j * L, L)] = z16i

    def start_copy(chunk_idx, buf, sem):
        off = jnp.minimum(word0 + chunk_idx * CW, last_off)
        return pltpu.async_copy(x_hbm.at[pl.ds(off, CW)], buf, sem)

    start_copy(0, buf0, sem0)

    def pair_body(pi, ps):
        ps = list(ps)
        start_copy(2 * pi + 1, buf1, sem1)
        pltpu.make_async_copy(x_hbm.at[pl.ds(0, CW)], buf0, sem0).wait()
        ps = process(buf0, 2 * pi * CHUNK, ps)
        start_copy(2 * pi + 2, buf0, sem0)
        pltpu.make_async_copy(x_hbm.at[pl.ds(0, CW)], buf1, sem1).wait()
        ps = process(buf1, (2 * pi + 1) * CHUNK, ps)
        return tuple(ps)

    ps = lax.fori_loop(0, NPAIRS, pair_body, (z16f,) * NV)
    pltpu.make_async_copy(x_hbm.at[pl.ds(0, CW)], buf0, sem0).wait()

    for j in range(NV):
        psum_buf[pl.ds(j * L, L)] = ps[j]

    pltpu.sync_copy(sample_buf, sample_hbm.at[pl.ds(tok0, TPW)])
    pltpu.sync_copy(mult_buf, mult_hbm.at[pl.ds(tok0, TPW)])

    pltpu.sync_copy(psum_buf, sh_ps.at[pl.ds(sid * NE, NE)])
    pltpu.sync_copy(cnt_buf, sh_ct.at[pl.ds(sid * NE, NE)])
    plsc.subcore_barrier()

    @pl.when(sid == 0)
    def _():
        pltpu.sync_copy(sh_ps, agg_ps)
        pltpu.sync_copy(sh_ct, agg_ct)
        accp = [z16f for _ in range(NV)]
        accc = [z16i for _ in range(NV)]
        for rr in range(NS):
            for j in range(NV):
                accp[j] = accp[j] + agg_ps[pl.ds(rr * NE + j * L, L)]
                accc[j] = accc[j] + agg_ct[pl.ds(rr * NE + j * L, L)]
        for j in range(NV):
            psum_buf[pl.ds(j * L, L)] = accp[j]
            cnt_buf[pl.ds(j * L, L)] = accc[j]
        pltpu.sync_copy(psum_buf, psum_hbm.at[pl.ds(cid * NE, NE)])
        pltpu.sync_copy(cnt_buf, cnt_hbm.at[pl.ds(cid * NE, NE)])


def _loss_body(cnt_ref, ps_ref, out_ref):
    cntf = cnt_ref[...].astype(jnp.float32)
    ps = ps_ref[...]
    f2 = jnp.sum(cntf, axis=0, keepdims=True) * (1.0 / NT)
    pm2 = jnp.sum(ps, axis=0, keepdims=True) * (1.0 / NT)
    out_ref[...] = jnp.float32(NE) * jnp.sum(pm2 * f2, axis=1, keepdims=True)


def kernel(logits):
    x = logits.reshape(-1)
    sample, mult, cnt, psum = _gate_kernel(x)
    loss = pl.pallas_call(
        _loss_body,
        out_shape=jax.ShapeDtypeStruct((1, 1), jnp.float32),
    )(cnt.reshape(NC, NE), psum.reshape(NC, NE))
    return sample, mult.reshape(NT, 1), loss.reshape(())


# speedup vs baseline: 1.3208x; 1.0109x over previous
import functools

import jax
import jax.numpy as jnp
from jax import lax
from jax.experimental import pallas as pl
from jax.experimental.pallas import tpu as pltpu
from jax.experimental.pallas import tpu_sc as plsc

NT = 32768
NE = 64
NC = 2
NS = 16
NW = NC * NS
TPW = NT // NW
CHUNK = 64
NPAIRS = TPW // (2 * CHUNK)
L = 16
NV = NE // L
CW = CHUNK * NE

_mesh = plsc.VectorSubcoreMesh(core_axis_name="c", subcore_axis_name="s")


@functools.partial(
    pl.kernel,
    out_type=[
        jax.ShapeDtypeStruct((NT,), jnp.int32),
        jax.ShapeDtypeStruct((NT,), jnp.float32),
        jax.ShapeDtypeStruct((NC * NE,), jnp.int32),
        jax.ShapeDtypeStruct((NC * NE,), jnp.float32),
    ],
    mesh=_mesh,
    compiler_params=pltpu.CompilerParams(needs_layout_passes=False),
    scratch_types=[
        pltpu.VMEM((TPW * NE,), jnp.float32),
        pltpu.VMEM((TPW,), jnp.int32),
        pltpu.VMEM((TPW,), jnp.float32),
        pltpu.VMEM((NE,), jnp.int32),
        pltpu.VMEM((NE,), jnp.float32),
        pltpu.VMEM((NS * NE,), jnp.float32),
        pltpu.VMEM((NS * NE,), jnp.int32),
        pltpu.VMEM_SHARED((NS * NE,), jnp.float32),
        pltpu.VMEM_SHARED((NS * NE,), jnp.int32),
        pltpu.SemaphoreType.DMA,
        pltpu.SemaphoreType.DMA,
    ],
)
def _gate_kernel(x_hbm, sample_hbm, mult_hbm, cnt_hbm, psum_hbm,
                 big, sample_buf, mult_buf, cnt_buf, psum_buf,
                 agg_ps, agg_ct, sh_ps, sh_ct, sem0, sem1):
    cid = lax.axis_index("c")
    sid = lax.axis_index("s")
    wid = cid * NS + sid
    tok0 = wid * TPW
    word0 = tok0 * NE
    last_off = word0 + TPW * NE - CW

    idx0 = lax.iota(jnp.int32, L)
    ones_i = jnp.ones((L,), jnp.int32)
    z16f = jnp.zeros((L,), jnp.float32)
    z16i = jnp.zeros((L,), jnp.int32)
    idxc = [idx0 + j * L for j in range(NV)]
    lmask = [idx0 == t for t in range(L)]
    big_i = jnp.full((L,), NE, jnp.int32)

    def process(out_off, ps):
        ps = list(ps)
        for blk in range(CHUNK // L):
            svec = z16i
            mvec = z16f
            for tt in range(L):
                t = blk * L + tt
                l = [big[pl.ds((out_off + t) * NE + j * L, L)]
                     for j in range(NV)]
                m = jnp.max(jnp.maximum(jnp.maximum(l[0], l[1]),
                                        jnp.maximum(l[2], l[3])))
                ex = [jnp.exp(l[j] - m) for j in range(NV)]
                s = jnp.sum((ex[0] + ex[1]) + (ex[2] + ex[3]))
                r = 1.0 / jnp.broadcast_to(s, (L,))
                c = [jnp.where(l[j] == m, idxc[j], big_i) for j in range(NV)]
                samp = jnp.min(jnp.minimum(jnp.minimum(c[0], c[1]),
                                           jnp.minimum(c[2], c[3])))
                for j in range(NV):
                    ps[j] = ps[j] + ex[j] * r
                svec = jnp.where(lmask[tt], samp, svec)
                mvec = jnp.where(lmask[tt], r, mvec)
            sample_buf[pl.ds(out_off + blk * L, L)] = svec
            mult_buf[pl.ds(out_off + blk * L, L)] = mvec
            plsc.addupdate_scatter(cnt_buf, [svec], ones_i)
        return ps

    for j in range(NV):
        cnt_buf[pl.ds(j * L, L)] = z16i

    HW = TPW * NE // 2
    HT = TPW // 2
    cp0 = pltpu.async_copy(x_hbm.at[pl.ds(word0, HW)],
                           big.at[pl.ds(0, HW)], sem0)
    cp1 = pltpu.async_copy(x_hbm.at[pl.ds(word0 + HW, HW)],
                           big.at[pl.ds(HW, HW)], sem1)

    ps = (z16f,) * NV
    for h, cp in ((0, cp0), (1, cp1)):
        cp.wait()

        def chunk_body(ci, ps, h=h):
            return tuple(process(h * HT + ci * CHUNK, list(ps)))

        ps = lax.fori_loop(0, HT // CHUNK, chunk_body, ps)

    for j in range(NV):
        psum_buf[pl.ds(j * L, L)] = ps[j]

    pltpu.sync_copy(sample_buf, sample_hbm.at[pl.ds(tok0, TPW)])
    pltpu.sync_copy(mult_buf, mult_hbm.at[pl.ds(tok0, TPW)])

    pltpu.sync_copy(psum_buf, sh_ps.at[pl.ds(sid * NE, NE)])
    pltpu.sync_copy(cnt_buf, sh_ct.at[pl.ds(sid * NE, NE)])
    plsc.subcore_barrier()

    @pl.when(sid == 0)
    def _():
        pltpu.sync_copy(sh_ps, agg_ps)
        pltpu.sync_copy(sh_ct, agg_ct)
        accp = [z16f for _ in range(NV)]
        accc = [z16i for _ in range(NV)]
        for rr in range(NS):
            for j in range(NV):
                accp[j] = accp[j] + agg_ps[pl.ds(rr * NE + j * L, L)]
                accc[j] = accc[j] + agg_ct[pl.ds(rr * NE + j * L, L)]
        for j in range(NV):
            psum_buf[pl.ds(j * L, L)] = accp[j]
            cnt_buf[pl.ds(j * L, L)] = accc[j]
        pltpu.sync_copy(psum_buf, psum_hbm.at[pl.ds(cid * NE, NE)])
        pltpu.sync_copy(cnt_buf, cnt_hbm.at[pl.ds(cid * NE, NE)])


def _loss_body(cnt_ref, ps_ref, out_ref):
    cntf = cnt_ref[...].astype(jnp.float32)
    ps = ps_ref[...]
    f2 = jnp.sum(cntf, axis=0, keepdims=True) * (1.0 / NT)
    pm2 = jnp.sum(ps, axis=0, keepdims=True) * (1.0 / NT)
    out_ref[...] = jnp.float32(NE) * jnp.sum(pm2 * f2, axis=1, keepdims=True)


def kernel(logits):
    x = logits.reshape(-1)
    sample, mult, cnt, psum = _gate_kernel(x)
    loss = pl.pallas_call(
        _loss_body,
        out_shape=jax.ShapeDtypeStruct((1, 1), jnp.float32),
    )(cnt.reshape(NC, NE), psum.reshape(NC, NE))
    return sample, mult.reshape(NT, 1), loss.reshape(())
